# baseline (speedup 1.0000x reference)
import jax
import jax.numpy as jnp
from jax.experimental import pallas as pl
from jax.sharding import PartitionSpec as P

_N = 4096
_D = 512
_G = 2
_BM = 512
_ALPHA = 0.1


def _layer0_body(a_ref, xf_ref, xl_ref, w16_ref, o_ref):
    a16 = a_ref[0].astype(jnp.bfloat16)
    t = jnp.dot(a16, xf_ref[0], preferred_element_type=jnp.float32)
    t = (1.0 - _ALPHA) * t + _ALPHA * xl_ref[0].astype(jnp.float32)
    h = jnp.dot(t.astype(jnp.bfloat16), w16_ref[0],
                preferred_element_type=jnp.float32)
    o_ref[0] = jnp.maximum(h, 0.0).astype(jnp.bfloat16)


def _layer1_body(a_ref, hf_ref, hl_ref, w16_ref, o_ref):
    j = pl.program_id(1)
    a16 = a_ref[0].astype(jnp.bfloat16)
    t = jnp.dot(a16, hf_ref[j], preferred_element_type=jnp.float32)
    t = (1.0 - _ALPHA) * t + _ALPHA * hl_ref[0].astype(jnp.float32)
    c = jnp.dot(t.astype(jnp.bfloat16), w16_ref[j],
                preferred_element_type=jnp.float32) * (1.0 / _G)

    @pl.when(j == 0)
    def _():
        o_ref[...] = c

    @pl.when(j > 0)
    def _():
        o_ref[...] += c


def _forward_local(a, x16_full, x16_loc, w0_16, w1_16):
    nl = a.shape[1]
    rl = nl // _BM
    h1_loc = pl.pallas_call(
        _layer0_body,
        grid=(_G, rl),
        in_specs=[
            pl.BlockSpec((1, _BM, _N), lambda j, r: (j, r, 0)),
            pl.BlockSpec((1, _N, _D), lambda j, r: (j, 0, 0)),
            pl.BlockSpec((1, _BM, _D), lambda j, r: (j, r, 0)),
            pl.BlockSpec((1, _D, _D), lambda j, r: (j, 0, 0)),
        ],
        out_specs=pl.BlockSpec((1, _BM, _D), lambda j, r: (j, r, 0)),
        out_shape=jax.ShapeDtypeStruct((_G, nl, _D), jnp.bfloat16),
    )(a, x16_full, x16_loc, w0_16)
    return h1_loc


def _layer1_local(a, h16_full, h16_loc, w1_16):
    nl = a.shape[1]
    rl = nl // _BM
    return pl.pallas_call(
        _layer1_body,
        grid=(rl, _G),
        in_specs=[
            pl.BlockSpec((1, _BM, _N), lambda r, j: (j, r, 0)),
            pl.BlockSpec((_G, _N, _D), lambda r, j: (0, 0, 0)),
            pl.BlockSpec((1, _BM, _D), lambda r, j: (j, r, 0)),
            pl.BlockSpec((_G, _D, _D), lambda r, j: (0, 0, 0)),
        ],
        out_specs=pl.BlockSpec((_BM, _D), lambda r, j: (r, 0)),
        out_shape=jax.ShapeDtypeStruct((nl, _D), jnp.float32),
    )(a, h16_full, h16_loc, w1_16)


def kernel(adj_list, x_list, W_0_0, W_0_1, W_1_0, W_1_1):
    x16 = x_list.astype(jnp.bfloat16)
    w0_16 = jnp.stack([W_0_0, W_0_1]).astype(jnp.bfloat16)
    w1_16 = jnp.stack([W_1_0, W_1_1]).astype(jnp.bfloat16)

    devs = jax.devices()
    if len(devs) >= 2:
        mesh = jax.make_mesh(
            (2,), ("d",), devices=devs[:2],
            axis_types=(jax.sharding.AxisType.Explicit,))
        nl = _N // 2

        def shard_fn(a, xf, w0, w1):
            idx = jax.lax.axis_index("d")
            xl = jax.lax.dynamic_slice_in_dim(xf, idx * nl, nl, axis=1)
            h1_loc = _forward_local(a, xf, xl, w0, w1)
            h1_full = jax.lax.all_gather(h1_loc, "d", axis=1, tiled=True)
            return _layer1_local(a, h1_full, h1_loc, w1)

        ns = lambda spec: jax.sharding.NamedSharding(mesh, spec)
        adj_sh = jax.reshard(adj_list, ns(P(None, "d", None)))
        x16_sh = jax.reshard(x16, ns(P()))
        w0_sh = jax.reshard(w0_16, ns(P()))
        w1_sh = jax.reshard(w1_16, ns(P()))
        return jax.shard_map(
            shard_fn,
            mesh=mesh,
            in_specs=(P(None, "d", None), P(), P(), P()),
            out_specs=P("d", None),
            check_vma=False,
        )(adj_sh, x16_sh, w0_sh, w1_sh)

    h1 = _forward_local(adj_list, x16, x16, w0_16, w1_16)
    return _layer1_local(adj_list, h1, h1, w1_16)


# speedup vs baseline: 4.5480x; 4.5480x over previous
import jax
import jax.numpy as jnp
from jax.experimental import pallas as pl

_N = 4096
_D = 512
_G = 2
_BM = 512
_ALPHA = 0.1


def _layer0_body(a_ref, xf_ref, xl_ref, w16_ref, o_ref):
    a16 = a_ref[0].astype(jnp.bfloat16)
    t = jnp.dot(a16, xf_ref[0], preferred_element_type=jnp.float32)
    t = (1.0 - _ALPHA) * t + _ALPHA * xl_ref[0].astype(jnp.float32)
    h = jnp.dot(t.astype(jnp.bfloat16), w16_ref[0],
                preferred_element_type=jnp.float32)
    o_ref[0] = jnp.maximum(h, 0.0).astype(jnp.bfloat16)


def _layer1_body(a_ref, hf_ref, hl_ref, w16_ref, o_ref):
    j = pl.program_id(1)
    a16 = a_ref[0].astype(jnp.bfloat16)
    t = jnp.dot(a16, hf_ref[j], preferred_element_type=jnp.float32)
    t = (1.0 - _ALPHA) * t + _ALPHA * hl_ref[0].astype(jnp.float32)
    c = jnp.dot(t.astype(jnp.bfloat16), w16_ref[j],
                preferred_element_type=jnp.float32) * (1.0 / _G)

    @pl.when(j == 0)
    def _():
        o_ref[...] = c

    @pl.when(j > 0)
    def _():
        o_ref[...] += c


def kernel(adj_list, x_list, W_0_0, W_0_1, W_1_0, W_1_1):
    x16 = x_list.astype(jnp.bfloat16)
    w0_16 = jnp.stack([W_0_0, W_0_1]).astype(jnp.bfloat16)
    w1_16 = jnp.stack([W_1_0, W_1_1]).astype(jnp.bfloat16)
    r = _N // _BM

    h16 = pl.pallas_call(
        _layer0_body,
        grid=(_G, r),
        in_specs=[
            pl.BlockSpec((1, _BM, _N), lambda j, r: (j, r, 0)),
            pl.BlockSpec((1, _N, _D), lambda j, r: (j, 0, 0)),
            pl.BlockSpec((1, _BM, _D), lambda j, r: (j, r, 0)),
            pl.BlockSpec((1, _D, _D), lambda j, r: (j, 0, 0)),
        ],
        out_specs=pl.BlockSpec((1, _BM, _D), lambda j, r: (j, r, 0)),
        out_shape=jax.ShapeDtypeStruct((_G, _N, _D), jnp.bfloat16),
    )(adj_list, x16, x16, w0_16)

    return pl.pallas_call(
        _layer1_body,
        grid=(r, _G),
        in_specs=[
            pl.BlockSpec((1, _BM, _N), lambda r, j: (j, r, 0)),
            pl.BlockSpec((_G, _N, _D), lambda r, j: (0, 0, 0)),
            pl.BlockSpec((1, _BM, _D), lambda r, j: (j, r, 0)),
            pl.BlockSpec((_G, _D, _D), lambda r, j: (0, 0, 0)),
        ],
        out_specs=pl.BlockSpec((_BM, _D), lambda r, j: (r, 0)),
        out_shape=jax.ShapeDtypeStruct((_N, _D), jnp.float32),
    )(adj_list, h16, h16, w1_16)
